# Initial kernel scaffold; baseline (speedup 1.0000x reference)
import jax
import jax.numpy as jnp
from jax.experimental import pallas as pl


def kernel(x, edge_index, W1_l, b1, W1_r, W2_l, b2, W2_r):
    raise NotImplementedError("write your pallas kernel here")


# speedup vs baseline: 7.2505x; 7.2505x over previous
import functools

import jax
import jax.numpy as jnp
from jax import lax
from jax.experimental import pallas as pl
from jax.experimental.pallas import tpu as pltpu
from jax.experimental.pallas import tpu_sc as plsc

NC = 2
NS = 16
NW = NC * NS
CHUNK = 80


def _dense_pre(x, w_l, w_r, b):
    n, d = x.shape

    def body(x_ref, wl_ref, wr_ref, b_ref, y_ref, z_ref):
        xv = x_ref[...]
        y_ref[...] = jnp.dot(xv, wl_ref[...], preferred_element_type=jnp.float32)
        z_ref[...] = (
            jnp.dot(xv, wr_ref[...], preferred_element_type=jnp.float32)
            + b_ref[...]
        )

    sds = jax.ShapeDtypeStruct((n, d), jnp.float32)
    return pl.pallas_call(body, out_shape=(sds, sds))(x, w_l, w_r, b)


def _dense_mid(p0, p1, z1, w_l, w_r, b):
    n, d = p0.shape

    def body(p0_ref, p1_ref, z1_ref, wl_ref, wr_ref, b_ref, h_ref, y_ref, z_ref):
        h = jnp.maximum(p0_ref[...] + p1_ref[...] + z1_ref[...], 0.0)
        h_ref[...] = h
        y_ref[...] = jnp.dot(h, wl_ref[...], preferred_element_type=jnp.float32)
        z_ref[...] = (
            jnp.dot(h, wr_ref[...], preferred_element_type=jnp.float32)
            + b_ref[...]
        )

    sds = jax.ShapeDtypeStruct((n, d), jnp.float32)
    return pl.pallas_call(body, out_shape=(sds, sds, sds))(p0, p1, z1, w_l, w_r, b)


def _dense_post(q0, q1, z2):
    n, d = q0.shape

    def body(q0_ref, q1_ref, z2_ref, o_ref):
        o_ref[...] = q0_ref[...] + q1_ref[...] + z2_ref[...]

    return pl.pallas_call(
        body, out_shape=jax.ShapeDtypeStruct((n, d), jnp.float32)
    )(q0, q1, z2)


def _edge_aggregate(y, src3, dst3, zeros):
    n, d = y.shape
    n_chunks = src3.shape[1]
    rows_main = (n // NS) // 8 * 8
    tail = n - NS * rows_main
    tail_off = NS * rows_main

    mesh = plsc.VectorSubcoreMesh(
        core_axis_name="c", subcore_axis_name="s", num_cores=NC, num_subcores=NS
    )
    sds = jax.ShapeDtypeStruct((n, d), jnp.float32)

    @functools.partial(
        pl.kernel,
        out_type=(sds, sds),
        mesh=mesh,
        scratch_types=[
            pltpu.VMEM((n_chunks, CHUNK), jnp.int32),
            pltpu.VMEM((n_chunks, CHUNK), jnp.int32),
            pltpu.VMEM((CHUNK, d), jnp.float32),
            pltpu.VMEM_SHARED((n, d), jnp.float32),
            pltpu.SemaphoreType.DMA,
        ],
    )
    def k(y_hbm, src_hbm, dst_hbm, z_hbm, out0, out1, srcv, dstv, rowsv, acc, sem):
        c = lax.axis_index("c")
        s = lax.axis_index("s")
        wid = c * NS + s
        pltpu.sync_copy(src_hbm.at[wid], srcv)
        pltpu.sync_copy(dst_hbm.at[wid], dstv)
        r0 = s * rows_main
        pltpu.sync_copy(z_hbm.at[pl.ds(0, rows_main)], acc.at[pl.ds(r0, rows_main)])
        if tail:
            @pl.when(s == NS - 1)
            def _():
                pltpu.sync_copy(z_hbm.at[pl.ds(0, tail)], acc.at[pl.ds(tail_off, tail)])
        plsc.subcore_barrier()

        def chunk(i, carry):
            pltpu.async_copy(y_hbm.at[srcv.at[i]], rowsv, sem).wait()
            pltpu.sync_copy(rowsv, acc.at[dstv.at[i]], add=True)
            return carry

        lax.fori_loop(0, n_chunks, chunk, 0)
        plsc.subcore_barrier()

        @pl.when(c == 0)
        def _():
            pltpu.sync_copy(acc.at[pl.ds(r0, rows_main)], out0.at[pl.ds(r0, rows_main)])
            if tail:
                @pl.when(s == NS - 1)
                def _():
                    pltpu.sync_copy(acc.at[pl.ds(tail_off, tail)], out0.at[pl.ds(tail_off, tail)])

        @pl.when(c == 1)
        def _():
            pltpu.sync_copy(acc.at[pl.ds(r0, rows_main)], out1.at[pl.ds(r0, rows_main)])
            if tail:
                @pl.when(s == NS - 1)
                def _():
                    pltpu.sync_copy(acc.at[pl.ds(tail_off, tail)], out1.at[pl.ds(tail_off, tail)])

    return k(y, src3, dst3, zeros)


def kernel(x, edge_index, W1_l, b1, W1_r, W2_l, b2, W2_r):
    n, d = x.shape
    e = edge_index.shape[1]
    assert e % (NW * CHUNK) == 0 and n % NS == 0
    n_chunks = e // (NW * CHUNK)

    src3 = edge_index[0].reshape(NW, n_chunks, CHUNK)
    dst3 = edge_index[1].reshape(NW, n_chunks, CHUNK)
    zeros = jnp.zeros((n // NS, d), jnp.float32)
    b1r = b1.reshape(1, d)
    b2r = b2.reshape(1, d)

    y1, z1 = _dense_pre(x, W1_l, W1_r, b1r)
    p0, p1 = _edge_aggregate(y1, src3, dst3, zeros)
    h, y2, z2 = _dense_mid(p0, p1, z1, W2_l, W2_r, b2r)
    q0, q1 = _edge_aggregate(y2, src3, dst3, zeros)
    return _dense_post(q0, q1, z2)


# speedup vs baseline: 13.8325x; 1.9078x over previous
import functools

import jax
import jax.numpy as jnp
from jax import lax
from jax.experimental import pallas as pl
from jax.experimental.pallas import tpu as pltpu
from jax.experimental.pallas import tpu_sc as plsc

NC = 2
NS = 16
NW = NC * NS
CHUNK = 80
NBUF = 4
GC = 8


def _dense_pre(x, w_l, w_r, b):
    n, d = x.shape

    def body(x_ref, wl_ref, wr_ref, b_ref, y_ref, z_ref):
        xv = x_ref[...]
        y_ref[...] = jnp.dot(xv, wl_ref[...], preferred_element_type=jnp.float32)
        z_ref[...] = (
            jnp.dot(xv, wr_ref[...], preferred_element_type=jnp.float32)
            + b_ref[...]
        )

    sds = jax.ShapeDtypeStruct((n, d), jnp.float32)
    return pl.pallas_call(body, out_shape=(sds, sds))(x, w_l, w_r, b)


def _dense_mid(p0, p1, z1, w_l, w_r, b):
    n, d = p0.shape

    def body(p0_ref, p1_ref, z1_ref, wl_ref, wr_ref, b_ref, h_ref, y_ref, z_ref):
        h = jnp.maximum(p0_ref[...] + p1_ref[...] + z1_ref[...], 0.0)
        h_ref[...] = h
        y_ref[...] = jnp.dot(h, wl_ref[...], preferred_element_type=jnp.float32)
        z_ref[...] = (
            jnp.dot(h, wr_ref[...], preferred_element_type=jnp.float32)
            + b_ref[...]
        )

    sds = jax.ShapeDtypeStruct((n, d), jnp.float32)
    return pl.pallas_call(body, out_shape=(sds, sds, sds))(p0, p1, z1, w_l, w_r, b)


def _dense_post(q0, q1, z2):
    n, d = q0.shape

    def body(q0_ref, q1_ref, z2_ref, o_ref):
        o_ref[...] = q0_ref[...] + q1_ref[...] + z2_ref[...]

    return pl.pallas_call(
        body, out_shape=jax.ShapeDtypeStruct((n, d), jnp.float32)
    )(q0, q1, z2)


def _edge_aggregate(y, src3, dst3, zeros, n_chunks):
    n, d = y.shape
    n_rows = src3.shape[1]
    rows_main = (n // NS) // 8 * 8
    tail = n - NS * rows_main
    tail_off = NS * rows_main

    mesh = plsc.VectorSubcoreMesh(
        core_axis_name="c", subcore_axis_name="s", num_cores=NC, num_subcores=NS
    )
    sds = jax.ShapeDtypeStruct((n, d), jnp.float32)

    nbuf = NBUF
    gfull = n_chunks // GC
    rem = n_chunks - gfull * GC
    assert n_rows == gfull * GC + GC
    assert gfull % 3 == 0 and rem > nbuf - 1 and GC % nbuf == 0

    @functools.partial(
        pl.kernel,
        out_type=(sds, sds),
        mesh=mesh,
        scratch_types=[pltpu.VMEM((GC, CHUNK), jnp.int32) for _ in range(6)]
        + [pltpu.VMEM((CHUNK, d), jnp.float32) for _ in range(nbuf)]
        + [pltpu.VMEM_SHARED((n, d), jnp.float32)]
        + [pltpu.SemaphoreType.DMA for _ in range(3 + nbuf)],
    )
    def k(y_hbm, src_hbm, dst_hbm, z_hbm, out0, out1,
          sg0, sg1, sg2, dg0, dg1, dg2, r0v, r1v, r2v, r3v, acc,
          gs0, gs1, gs2, bs0, bs1, bs2, bs3):
        sgs = (sg0, sg1, sg2)
        dgs = (dg0, dg1, dg2)
        rbufs = (r0v, r1v, r2v, r3v)
        semg = (gs0, gs1, gs2)
        semr = (bs0, bs1, bs2, bs3)
        c = lax.axis_index("c")
        s = lax.axis_index("s")
        wid = c * NS + s

        def fetch_group(g, p):
            off = pl.multiple_of(g * GC, GC)
            pltpu.async_copy(src_hbm.at[wid, pl.ds(off, GC)], sgs[p], semg[p])
            pltpu.async_copy(dst_hbm.at[wid, pl.ds(off, GC)], dgs[p], semg[p])

        def wait_group(g, p):
            off = pl.multiple_of(g * GC, GC)
            pltpu.make_async_copy(src_hbm.at[wid, pl.ds(off, GC)], sgs[p], semg[p]).wait()
            pltpu.make_async_copy(dst_hbm.at[wid, pl.ds(off, GC)], dgs[p], semg[p]).wait()

        def fire_gather(p, j, b):
            pltpu.async_copy(y_hbm.at[sgs[p].at[j]], rbufs[b], semr[b])

        def wait_gather(p, j, b):
            pltpu.make_async_copy(y_hbm.at[sgs[p].at[j]], rbufs[b], semr[b]).wait()

        r0 = s * rows_main
        pltpu.sync_copy(z_hbm.at[pl.ds(0, rows_main)], acc.at[pl.ds(r0, rows_main)])
        if tail:
            @pl.when(s == NS - 1)
            def _():
                pltpu.sync_copy(z_hbm.at[pl.ds(0, tail)], acc.at[pl.ds(tail_off, tail)])
        plsc.subcore_barrier()

        fetch_group(0, 0)
        fetch_group(1, 1)
        wait_group(0, 0)
        for j in range(nbuf):
            fire_gather(0, j, j)

        def triple(t, carry):
            for gk in range(3):
                g = t * 3 + gk
                p = gk
                pn = (gk + 1) % 3
                pf = (gk + 2) % 3
                wait_group(g + 1, pn)

                @pl.when(g + 2 < gfull + 1)
                def _(g=g, pf=pf):
                    fetch_group(g + 2, pf)

                for j in range(GC):
                    i = g * GC + j
                    b = j % nbuf
                    wait_gather(p, j, b)
                    pltpu.sync_copy(rbufs[b], acc.at[dgs[p].at[j]], add=True)
                    jn = j + nbuf
                    if jn < GC:
                        fire_gather(p, jn, b)
                    else:
                        fire_gather(pn, jn - GC, b)
            return carry

        lax.fori_loop(0, gfull // 3, triple, 0)

        for j in range(rem):
            b = j % nbuf
            wait_gather(0, j, b)
            pltpu.sync_copy(rbufs[b], acc.at[dgs[0].at[j]], add=True)
            jn = j + nbuf
            if jn < rem:
                fire_gather(0, jn, b)
        plsc.subcore_barrier()

        @pl.when(c == 0)
        def _():
            pltpu.sync_copy(acc.at[pl.ds(r0, rows_main)], out0.at[pl.ds(r0, rows_main)])
            if tail:
                @pl.when(s == NS - 1)
                def _():
                    pltpu.sync_copy(acc.at[pl.ds(tail_off, tail)], out0.at[pl.ds(tail_off, tail)])

        @pl.when(c == 1)
        def _():
            pltpu.sync_copy(acc.at[pl.ds(r0, rows_main)], out1.at[pl.ds(r0, rows_main)])
            if tail:
                @pl.when(s == NS - 1)
                def _():
                    pltpu.sync_copy(acc.at[pl.ds(tail_off, tail)], out1.at[pl.ds(tail_off, tail)])

    return k(y, src3, dst3, zeros)


def kernel(x, edge_index, W1_l, b1, W1_r, W2_l, b2, W2_r):
    n, d = x.shape
    e = edge_index.shape[1]
    assert e % (NW * CHUNK) == 0 and n % NS == 0
    n_chunks = e // (NW * CHUNK)
    pad_rows = GC - n_chunks % GC

    src3 = edge_index[0].reshape(NW, n_chunks, CHUNK)
    dst3 = edge_index[1].reshape(NW, n_chunks, CHUNK)
    idx_pad = jnp.zeros((NW, pad_rows, CHUNK), jnp.int32)
    src3 = jnp.concatenate([src3, idx_pad], axis=1)
    dst3 = jnp.concatenate([dst3, idx_pad], axis=1)
    zeros = jnp.zeros((n // NS, d), jnp.float32)
    b1r = b1.reshape(1, d)
    b2r = b2.reshape(1, d)

    y1, z1 = _dense_pre(x, W1_l, W1_r, b1r)
    p0, p1 = _edge_aggregate(y1, src3, dst3, zeros, n_chunks)
    h, y2, z2 = _dense_mid(p0, p1, z1, W2_l, W2_r, b2r)
    q0, q1 = _edge_aggregate(y2, src3, dst3, zeros, n_chunks)
    return _dense_post(q0, q1, z2)


# speedup vs baseline: 13.8698x; 1.0027x over previous
import functools

import jax
import jax.numpy as jnp
from jax import lax
from jax.experimental import pallas as pl
from jax.experimental.pallas import tpu as pltpu
from jax.experimental.pallas import tpu_sc as plsc

NC = 2
NS = 16
NW = NC * NS
CHUNK = 80
NBUF = 4
GC = 8


def _matmul(x, w):
    n, d = x.shape

    def body(x_ref, w_ref, y_ref):
        y_ref[...] = jnp.dot(x_ref[...], w_ref[...], preferred_element_type=jnp.float32)

    return pl.pallas_call(
        body, out_shape=jax.ShapeDtypeStruct((n, d), jnp.float32)
    )(x, w)


def _matmul_bias(x, w, b):
    n, d = x.shape

    def body(x_ref, w_ref, b_ref, z_ref):
        z_ref[...] = (
            jnp.dot(x_ref[...], w_ref[...], preferred_element_type=jnp.float32)
            + b_ref[...]
        )

    return pl.pallas_call(
        body, out_shape=jax.ShapeDtypeStruct((n, d), jnp.float32)
    )(x, w, b)


def _dense_mid(p0, p1, z1, w_l):
    n, d = p0.shape

    def body(p0_ref, p1_ref, z1_ref, wl_ref, h_ref, y_ref):
        h = jnp.maximum(p0_ref[...] + p1_ref[...] + z1_ref[...], 0.0)
        h_ref[...] = h
        y_ref[...] = jnp.dot(h, wl_ref[...], preferred_element_type=jnp.float32)

    sds = jax.ShapeDtypeStruct((n, d), jnp.float32)
    return pl.pallas_call(body, out_shape=(sds, sds))(p0, p1, z1, w_l)


def _dense_post(q0, q1, z2):
    n, d = q0.shape

    def body(q0_ref, q1_ref, z2_ref, o_ref):
        o_ref[...] = q0_ref[...] + q1_ref[...] + z2_ref[...]

    return pl.pallas_call(
        body, out_shape=jax.ShapeDtypeStruct((n, d), jnp.float32)
    )(q0, q1, z2)


def _edge_aggregate(y, src3, dst3, zeros, n_chunks):
    n, d = y.shape
    n_rows = src3.shape[1]
    rows_main = (n // NS) // 8 * 8
    tail = n - NS * rows_main
    tail_off = NS * rows_main

    mesh = plsc.VectorSubcoreMesh(
        core_axis_name="c", subcore_axis_name="s", num_cores=NC, num_subcores=NS
    )
    sds = jax.ShapeDtypeStruct((n, d), jnp.float32)

    nbuf = NBUF
    gfull = n_chunks // GC
    rem = n_chunks - gfull * GC
    assert n_rows == gfull * GC + GC
    assert gfull % 3 == 0 and rem > nbuf - 1 and GC % nbuf == 0

    @functools.partial(
        pl.kernel,
        out_type=(sds, sds),
        mesh=mesh,
        scratch_types=[pltpu.VMEM((GC, CHUNK), jnp.int32) for _ in range(6)]
        + [pltpu.VMEM((CHUNK, d), jnp.float32) for _ in range(nbuf)]
        + [pltpu.VMEM_SHARED((n, d), jnp.float32)]
        + [pltpu.SemaphoreType.DMA for _ in range(3 + nbuf)],
    )
    def k(y_hbm, src_hbm, dst_hbm, z_hbm, out0, out1,
          sg0, sg1, sg2, dg0, dg1, dg2, r0v, r1v, r2v, r3v, acc,
          gs0, gs1, gs2, bs0, bs1, bs2, bs3):
        sgs = (sg0, sg1, sg2)
        dgs = (dg0, dg1, dg2)
        rbufs = (r0v, r1v, r2v, r3v)
        semg = (gs0, gs1, gs2)
        semr = (bs0, bs1, bs2, bs3)
        c = lax.axis_index("c")
        s = lax.axis_index("s")
        wid = c * NS + s

        def fetch_group(g, p):
            off = pl.multiple_of(g * GC, GC)
            pltpu.async_copy(src_hbm.at[wid, pl.ds(off, GC)], sgs[p], semg[p])
            pltpu.async_copy(dst_hbm.at[wid, pl.ds(off, GC)], dgs[p], semg[p])

        def wait_group(g, p):
            off = pl.multiple_of(g * GC, GC)
            pltpu.make_async_copy(src_hbm.at[wid, pl.ds(off, GC)], sgs[p], semg[p]).wait()
            pltpu.make_async_copy(dst_hbm.at[wid, pl.ds(off, GC)], dgs[p], semg[p]).wait()

        def fire_gather(p, j, b):
            pltpu.async_copy(y_hbm.at[sgs[p].at[j]], rbufs[b], semr[b])

        def wait_gather(p, j, b):
            pltpu.make_async_copy(y_hbm.at[sgs[p].at[j]], rbufs[b], semr[b]).wait()

        r0 = s * rows_main
        pltpu.sync_copy(z_hbm.at[pl.ds(0, rows_main)], acc.at[pl.ds(r0, rows_main)])
        if tail:
            @pl.when(s == NS - 1)
            def _():
                pltpu.sync_copy(z_hbm.at[pl.ds(0, tail)], acc.at[pl.ds(tail_off, tail)])
        plsc.subcore_barrier()

        fetch_group(0, 0)
        fetch_group(1, 1)
        wait_group(0, 0)
        for j in range(nbuf):
            fire_gather(0, j, j)

        def triple(t, carry):
            for gk in range(3):
                g = t * 3 + gk
                p = gk
                pn = (gk + 1) % 3
                pf = (gk + 2) % 3
                wait_group(g + 1, pn)

                @pl.when(g + 2 < gfull + 1)
                def _(g=g, pf=pf):
                    fetch_group(g + 2, pf)

                for j in range(GC):
                    i = g * GC + j
                    b = j % nbuf
                    wait_gather(p, j, b)
                    pltpu.sync_copy(rbufs[b], acc.at[dgs[p].at[j]], add=True)
                    jn = j + nbuf
                    if jn < GC:
                        fire_gather(p, jn, b)
                    else:
                        fire_gather(pn, jn - GC, b)
            return carry

        lax.fori_loop(0, gfull // 3, triple, 0)

        for j in range(rem):
            b = j % nbuf
            wait_gather(0, j, b)
            pltpu.sync_copy(rbufs[b], acc.at[dgs[0].at[j]], add=True)
            jn = j + nbuf
            if jn < rem:
                fire_gather(0, jn, b)
        plsc.subcore_barrier()

        @pl.when(c == 0)
        def _():
            pltpu.sync_copy(acc.at[pl.ds(r0, rows_main)], out0.at[pl.ds(r0, rows_main)])
            if tail:
                @pl.when(s == NS - 1)
                def _():
                    pltpu.sync_copy(acc.at[pl.ds(tail_off, tail)], out0.at[pl.ds(tail_off, tail)])

        @pl.when(c == 1)
        def _():
            pltpu.sync_copy(acc.at[pl.ds(r0, rows_main)], out1.at[pl.ds(r0, rows_main)])
            if tail:
                @pl.when(s == NS - 1)
                def _():
                    pltpu.sync_copy(acc.at[pl.ds(tail_off, tail)], out1.at[pl.ds(tail_off, tail)])

    return k(y, src3, dst3, zeros)


def kernel(x, edge_index, W1_l, b1, W1_r, W2_l, b2, W2_r):
    n, d = x.shape
    e = edge_index.shape[1]
    assert e % (NW * CHUNK) == 0 and n % NS == 0
    n_chunks = e // (NW * CHUNK)
    pad_rows = GC - n_chunks % GC

    src3 = edge_index[0].reshape(NW, n_chunks, CHUNK)
    dst3 = edge_index[1].reshape(NW, n_chunks, CHUNK)
    idx_pad = jnp.zeros((NW, pad_rows, CHUNK), jnp.int32)
    src3 = jnp.concatenate([src3, idx_pad], axis=1)
    dst3 = jnp.concatenate([dst3, idx_pad], axis=1)
    zeros = jnp.zeros((n // NS, d), jnp.float32)
    b1r = b1.reshape(1, d)
    b2r = b2.reshape(1, d)

    y1 = _matmul(x, W1_l)
    p0, p1 = _edge_aggregate(y1, src3, dst3, zeros, n_chunks)
    z1 = _matmul_bias(x, W1_r, b1r)
    h, y2 = _dense_mid(p0, p1, z1, W2_l)
    q0, q1 = _edge_aggregate(y2, src3, dst3, zeros, n_chunks)
    z2 = _matmul_bias(h, W2_r, b2r)
    return _dense_post(q0, q1, z2)
